# baseline (speedup 1.0000x reference)
import functools
import math

import jax
import jax.numpy as jnp
from jax import lax
from jax.experimental import pallas as pl
from jax.experimental.pallas import tpu as pltpu
from jax.experimental.pallas import tpu_sc as plsc

_N_TOPIC = 256
_V_DIM = 10000
_HID = 2048
_EMB = 300
_VOCAB = 100000
_K = _N_TOPIC + _V_DIM

_BH = 512
_GRID = _HID // _BH
_BN_INV = 1.0 / math.sqrt(1.0 + 1e-5)


def _mlp_body(p_ref, w1t_ref, gamma_ref, beta_ref, b1_ref, w2_ref, b2_ref, out_ref):
    i = pl.program_id(0)
    h = lax.dot_general(
        p_ref[...], w1t_ref[...], (((1,), (0,)), ((), ())),
        preferred_element_type=jnp.float32,
    )
    scale = gamma_ref[...] * _BN_INV
    h = (h + b1_ref[...]) * scale + beta_ref[...]
    h = jnp.where(h > 0.0, h, 0.01 * h)
    part = jnp.sum(h * w2_ref[...], axis=1, keepdims=True)

    @pl.when(i == 0)
    def _init():
        out_ref[...] = b2_ref[...] + part

    @pl.when(i > 0)
    def _acc():
        out_ref[...] += part


_mlp_call = pl.pallas_call(
    _mlp_body,
    grid=(_GRID,),
    in_specs=[
        pl.BlockSpec((1, _K), lambda i: (0, 0)),
        pl.BlockSpec((_K, _BH), lambda i: (0, i)),
        pl.BlockSpec((1, _BH), lambda i: (0, i)),
        pl.BlockSpec((1, _BH), lambda i: (0, i)),
        pl.BlockSpec((1, _BH), lambda i: (0, i)),
        pl.BlockSpec((1, _BH), lambda i: (0, i)),
        pl.BlockSpec((1, 1), lambda i: (0, 0)),
    ],
    out_specs=pl.BlockSpec((1, 1), lambda i: (0, 0)),
    out_shape=jax.ShapeDtypeStruct((1, 1), jnp.float32),
)

_NW = 32
_BPW = 320
_PAD_N = _NW * _BPW
_GB = 80
_GCH = _BPW // _GB
_VPAD = 102400
_STRIPE = _VPAD // 16


def _scatter_body(idx_hbm, w_hbm, out_hbm, idx_v, w_v, zero_v, s_sh, sem):
    cid = lax.axis_index("c")
    sid = lax.axis_index("s")
    wid = sid * 2 + cid
    base = wid * _BPW
    for g in range(_GCH):
        pltpu.sync_copy(idx_hbm.at[pl.ds(base + g * _GB, _GB)], idx_v.at[g])
        pltpu.sync_copy(w_hbm.at[pl.ds(base + g * _GB, _GB)], w_v.at[g])

    def zloop(j, carry):
        zero_v[pl.ds(j * 16, 16)] = jnp.zeros((16,), jnp.float32)
        return carry

    lax.fori_loop(0, _STRIPE // 16, zloop, 0)
    pltpu.sync_copy(zero_v, s_sh.at[pl.ds(sid * _STRIPE, _STRIPE)])

    plsc.subcore_barrier()

    for g in range(_GCH):
        pltpu.sync_copy(w_v.at[g], s_sh.at[idx_v.at[g]], add=True)

    plsc.subcore_barrier()

    pltpu.sync_copy(s_sh.at[pl.ds(sid * _STRIPE, _STRIPE)],
                    out_hbm.at[cid, pl.ds(sid * _STRIPE, _STRIPE)])


@functools.lru_cache(maxsize=1)
def _build_scatter():
    return functools.partial(
        pl.kernel,
        out_type=jax.ShapeDtypeStruct((2, _VPAD), jnp.float32),
        mesh=plsc.VectorSubcoreMesh(core_axis_name="c", subcore_axis_name="s"),
        scratch_types=[
            pltpu.VMEM((_GCH, _GB), jnp.int32),
            pltpu.VMEM((_GCH, _GB), jnp.float32),
            pltpu.VMEM((_STRIPE,), jnp.float32),
            pltpu.VMEM_SHARED((_VPAD,), jnp.float32),
            pltpu.SemaphoreType.DMA,
        ],
    )(_scatter_body)


_BV = 12800
_VGRID = _VPAD // _BV


def _bag_body(s_ref, et_ref, out_ref):
    k = pl.program_id(0)
    sv = s_ref[0:1, :] + s_ref[1:2, :]
    last_valid = _VOCAB - (_VGRID - 1) * _BV

    @pl.when(k == _VGRID - 1)
    def _mask():
        col = lax.broadcasted_iota(jnp.int32, (_EMB, _BV), 1)
        et_ref[...] = jnp.where(col < last_valid, et_ref[...], 0.0)

    et = et_ref[...]
    part = lax.dot_general(
        sv, et, (((1,), (1,)), ((), ())),
        preferred_element_type=jnp.float32,
    )

    @pl.when(k == 0)
    def _init():
        out_ref[...] = part

    @pl.when(k > 0)
    def _acc():
        out_ref[...] += part


_bag_call = pl.pallas_call(
    _bag_body,
    grid=(_VGRID,),
    in_specs=[
        pl.BlockSpec((2, _BV), lambda k: (0, k)),
        pl.BlockSpec((_EMB, _BV), lambda k: (0, k)),
    ],
    out_specs=pl.BlockSpec((1, _EMB), lambda k: (0, 0)),
    out_shape=jax.ShapeDtypeStruct((1, _EMB), jnp.float32),
)


def kernel(theta, bow, word_inputs, W1, b1, gamma, beta, W2, b2, embedding):
    p = jnp.concatenate([theta, bow]).reshape(1, _K)
    score = _mlp_call(p, W1.T, gamma.reshape(1, _HID), beta.reshape(1, _HID),
                      b1.reshape(1, _HID), W2.reshape(1, _HID), b2.reshape(1, 1))

    pad = _PAD_N - _V_DIM
    idx_p = jnp.concatenate([word_inputs, jnp.zeros((pad,), jnp.int32)])
    w_p = jnp.concatenate([bow, jnp.zeros((pad,), jnp.float32)])
    s2 = _build_scatter()(idx_p, w_p)
    word_embedding = _bag_call(s2, embedding.T).reshape(_EMB)
    return score.reshape(1), word_embedding


# speedup vs baseline: 1.0253x; 1.0253x over previous
import functools
import math

import jax
import jax.numpy as jnp
from jax import lax
from jax.experimental import pallas as pl
from jax.experimental.pallas import tpu as pltpu
from jax.experimental.pallas import tpu_sc as plsc

_N_TOPIC = 256
_V_DIM = 10000
_HID = 2048
_EMB = 300
_VOCAB = 100000
_K = _N_TOPIC + _V_DIM

_BH = 512
_GRID = _HID // _BH
_BN_INV = 1.0 / math.sqrt(1.0 + 1e-5)


def _mlp_body(theta_ref, bow_ref, w1t_ref, gamma_ref, beta_ref, b1_ref, w2_ref,
              b2_ref, out_ref, p_scr):
    i = pl.program_id(0)

    @pl.when(i == 0)
    def _concat():
        p_scr[0:1, pl.ds(0, _N_TOPIC)] = theta_ref[...]
        p_scr[0:1, pl.ds(_N_TOPIC, _V_DIM)] = bow_ref[...]

    h = lax.dot_general(
        p_scr[...], w1t_ref[...], (((1,), (0,)), ((), ())),
        preferred_element_type=jnp.float32,
    )
    scale = gamma_ref[...] * _BN_INV
    h = (h + b1_ref[...]) * scale + beta_ref[...]
    h = jnp.where(h > 0.0, h, 0.01 * h)
    part = jnp.sum(h * w2_ref[...], axis=1, keepdims=True)

    @pl.when(i == 0)
    def _init():
        out_ref[...] = b2_ref[...] + part

    @pl.when(i > 0)
    def _acc():
        out_ref[...] += part


_mlp_call = pl.pallas_call(
    _mlp_body,
    grid=(_GRID,),
    in_specs=[
        pl.BlockSpec((1, _N_TOPIC), lambda i: (0, 0)),
        pl.BlockSpec((1, _V_DIM), lambda i: (0, 0)),
        pl.BlockSpec((_K, _BH), lambda i: (0, i)),
        pl.BlockSpec((1, _BH), lambda i: (0, i)),
        pl.BlockSpec((1, _BH), lambda i: (0, i)),
        pl.BlockSpec((1, _BH), lambda i: (0, i)),
        pl.BlockSpec((1, _BH), lambda i: (0, i)),
        pl.BlockSpec((1, 1), lambda i: (0, 0)),
    ],
    out_specs=pl.BlockSpec((1, 1), lambda i: (0, 0)),
    out_shape=jax.ShapeDtypeStruct((1, 1), jnp.float32),
    scratch_shapes=[pltpu.VMEM((1, _K), jnp.float32)],
)

_NW = 32
_BPW = 320
_PAD_N = _NW * _BPW
_GB = 80
_GCH = _BPW // _GB
_VPAD = 102400
_STRIPE = _VPAD // 16


def _scatter_body(idx_hbm, w_hbm, out_hbm, idx_v, w_v, zero_v, s_sh, sem):
    cid = lax.axis_index("c")
    sid = lax.axis_index("s")
    wid = sid * 2 + cid
    base = wid * _BPW

    def stage(g):
        pltpu.sync_copy(idx_hbm.at[pl.ds(base + g * _GB, _GB)], idx_v.at[g])
        pltpu.sync_copy(w_hbm.at[pl.ds(base + g * _GB, _GB)], w_v.at[g])

    stage(0)

    @pl.when(wid < _NW - 1)
    def _stage_rest():
        for g in range(1, _GCH):
            stage(g)

    def zloop(j, carry):
        zero_v[pl.ds(j * 16, 16)] = jnp.zeros((16,), jnp.float32)
        return carry

    lax.fori_loop(0, _STRIPE // 16, zloop, 0)
    pltpu.sync_copy(zero_v, s_sh.at[pl.ds(sid * _STRIPE, _STRIPE)])

    plsc.subcore_barrier()

    pltpu.sync_copy(w_v.at[0], s_sh.at[idx_v.at[0]], add=True)

    @pl.when(wid < _NW - 1)
    def _scatter_rest():
        for g in range(1, _GCH):
            pltpu.sync_copy(w_v.at[g], s_sh.at[idx_v.at[g]], add=True)

    plsc.subcore_barrier()

    pltpu.sync_copy(s_sh.at[pl.ds(sid * _STRIPE, _STRIPE)],
                    out_hbm.at[cid, pl.ds(sid * _STRIPE, _STRIPE)])


@functools.lru_cache(maxsize=1)
def _build_scatter():
    return functools.partial(
        pl.kernel,
        out_type=jax.ShapeDtypeStruct((2, _VPAD), jnp.float32),
        mesh=plsc.VectorSubcoreMesh(core_axis_name="c", subcore_axis_name="s"),
        scratch_types=[
            pltpu.VMEM((_GCH, _GB), jnp.int32),
            pltpu.VMEM((_GCH, _GB), jnp.float32),
            pltpu.VMEM((_STRIPE,), jnp.float32),
            pltpu.VMEM_SHARED((_VPAD,), jnp.float32),
            pltpu.SemaphoreType.DMA,
        ],
    )(_scatter_body)


_BV = 12800
_VGRID = _VPAD // _BV


def _bag_body(s_ref, et_ref, out_ref):
    k = pl.program_id(0)
    sv = s_ref[0:1, :] + s_ref[1:2, :]
    last_valid = _VOCAB - (_VGRID - 1) * _BV

    @pl.when(k == _VGRID - 1)
    def _mask():
        col = lax.broadcasted_iota(jnp.int32, (_EMB, _BV), 1)
        et_ref[...] = jnp.where(col < last_valid, et_ref[...], 0.0)

    et = et_ref[...]
    part = lax.dot_general(
        sv, et, (((1,), (1,)), ((), ())),
        preferred_element_type=jnp.float32,
    )

    @pl.when(k == 0)
    def _init():
        out_ref[...] = part

    @pl.when(k > 0)
    def _acc():
        out_ref[...] += part


_bag_call = pl.pallas_call(
    _bag_body,
    grid=(_VGRID,),
    in_specs=[
        pl.BlockSpec((2, _BV), lambda k: (0, k)),
        pl.BlockSpec((_EMB, _BV), lambda k: (0, k)),
    ],
    out_specs=pl.BlockSpec((1, _EMB), lambda k: (0, 0)),
    out_shape=jax.ShapeDtypeStruct((1, _EMB), jnp.float32),
)


def kernel(theta, bow, word_inputs, W1, b1, gamma, beta, W2, b2, embedding):
    score = _mlp_call(theta.reshape(1, _N_TOPIC), bow.reshape(1, _V_DIM), W1.T,
                      gamma.reshape(1, _HID), beta.reshape(1, _HID),
                      b1.reshape(1, _HID), W2.reshape(1, _HID), b2.reshape(1, 1))

    s2 = _build_scatter()(word_inputs, bow)
    word_embedding = _bag_call(s2, embedding.T).reshape(_EMB)
    return score.reshape(1), word_embedding


# speedup vs baseline: 1.0620x; 1.0358x over previous
import functools
import math

import jax
import jax.numpy as jnp
from jax import lax
from jax.experimental import pallas as pl
from jax.experimental.pallas import tpu as pltpu
from jax.experimental.pallas import tpu_sc as plsc

_N_TOPIC = 256
_V_DIM = 10000
_HID = 2048
_EMB = 300
_VOCAB = 100000
_K = _N_TOPIC + _V_DIM

_BH = 512
_GRID = _HID // _BH
_BN_INV = 1.0 / math.sqrt(1.0 + 1e-5)


def _mlp_body(theta_ref, bow_ref, w1t_ref, gamma_ref, beta_ref, b1_ref, w2_ref,
              b2_ref, out_ref, p_scr):
    i = pl.program_id(0)

    @pl.when(i == 0)
    def _concat():
        p_scr[0:1, pl.ds(0, _N_TOPIC)] = theta_ref[...][None, :]
        p_scr[0:1, pl.ds(_N_TOPIC, _V_DIM)] = bow_ref[...][None, :]

    h = lax.dot_general(
        p_scr[...], w1t_ref[...], (((1,), (0,)), ((), ())),
        preferred_element_type=jnp.float32,
    )
    sl = pl.ds(i * _BH, _BH)
    scale = gamma_ref[sl][None, :] * _BN_INV
    h = (h + b1_ref[sl][None, :]) * scale + beta_ref[sl][None, :]
    h = jnp.where(h > 0.0, h, 0.01 * h)
    part = jnp.sum(h * w2_ref[sl][None, :], axis=1, keepdims=True)

    @pl.when(i == 0)
    def _init():
        out_ref[...] = b2_ref[...][None, :] + part

    @pl.when(i > 0)
    def _acc():
        out_ref[...] += part


_mlp_call = pl.pallas_call(
    _mlp_body,
    grid=(_GRID,),
    in_specs=[
        pl.BlockSpec((_N_TOPIC,), lambda i: (0,)),
        pl.BlockSpec((_V_DIM,), lambda i: (0,)),
        pl.BlockSpec((_K, _BH), lambda i: (0, i)),
        pl.BlockSpec((_HID,), lambda i: (0,)),
        pl.BlockSpec((_HID,), lambda i: (0,)),
        pl.BlockSpec((_HID,), lambda i: (0,)),
        pl.BlockSpec((_HID,), lambda i: (0,)),
        pl.BlockSpec((1,), lambda i: (0,)),
    ],
    out_specs=pl.BlockSpec((1, 1), lambda i: (0, 0)),
    out_shape=jax.ShapeDtypeStruct((1, 1), jnp.float32),
    scratch_shapes=[pltpu.VMEM((1, _K), jnp.float32)],
)

_NW = 32
_BPW = 320
_PAD_N = _NW * _BPW
_GB = 80
_GCH = _BPW // _GB
_VPAD = 102400
_STRIPE = _VPAD // 16


def _scatter_body(idx_hbm, w_hbm, out_hbm, idx_v, w_v, zero_v, s_sh, sem):
    cid = lax.axis_index("c")
    sid = lax.axis_index("s")
    wid = sid * 2 + cid
    base = wid * _BPW

    def stage(g):
        pltpu.sync_copy(idx_hbm.at[pl.ds(base + g * _GB, _GB)], idx_v.at[g])
        pltpu.sync_copy(w_hbm.at[pl.ds(base + g * _GB, _GB)], w_v.at[g])

    stage(0)

    @pl.when(wid < _NW - 1)
    def _stage_rest():
        for g in range(1, _GCH):
            stage(g)

    def zloop(j, carry):
        zero_v[pl.ds(j * 16, 16)] = jnp.zeros((16,), jnp.float32)
        return carry

    lax.fori_loop(0, _STRIPE // 16, zloop, 0)
    pltpu.sync_copy(zero_v, s_sh.at[pl.ds(sid * _STRIPE, _STRIPE)])

    plsc.subcore_barrier()

    pltpu.sync_copy(w_v.at[0], s_sh.at[idx_v.at[0]], add=True)

    @pl.when(wid < _NW - 1)
    def _scatter_rest():
        for g in range(1, _GCH):
            pltpu.sync_copy(w_v.at[g], s_sh.at[idx_v.at[g]], add=True)

    plsc.subcore_barrier()

    pltpu.sync_copy(s_sh.at[pl.ds(sid * _STRIPE, _STRIPE)],
                    out_hbm.at[cid, pl.ds(sid * _STRIPE, _STRIPE)])


@functools.lru_cache(maxsize=1)
def _build_scatter():
    return functools.partial(
        pl.kernel,
        out_type=jax.ShapeDtypeStruct((2, _VPAD), jnp.float32),
        mesh=plsc.VectorSubcoreMesh(core_axis_name="c", subcore_axis_name="s"),
        scratch_types=[
            pltpu.VMEM((_GCH, _GB), jnp.int32),
            pltpu.VMEM((_GCH, _GB), jnp.float32),
            pltpu.VMEM((_STRIPE,), jnp.float32),
            pltpu.VMEM_SHARED((_VPAD,), jnp.float32),
            pltpu.SemaphoreType.DMA,
        ],
    )(_scatter_body)


_BV = 12800
_VGRID = _VPAD // _BV


def _bag_body(s_ref, et_ref, out_ref):
    k = pl.program_id(0)
    sv = s_ref[0:1, :] + s_ref[1:2, :]
    last_valid = _VOCAB - (_VGRID - 1) * _BV

    @pl.when(k == _VGRID - 1)
    def _mask():
        col = lax.broadcasted_iota(jnp.int32, (_EMB, _BV), 1)
        et_ref[...] = jnp.where(col < last_valid, et_ref[...], 0.0)

    et = et_ref[...]
    part = lax.dot_general(
        sv, et, (((1,), (1,)), ((), ())),
        preferred_element_type=jnp.float32,
    )

    @pl.when(k == 0)
    def _init():
        out_ref[...] = part

    @pl.when(k > 0)
    def _acc():
        out_ref[...] += part


_bag_call = pl.pallas_call(
    _bag_body,
    grid=(_VGRID,),
    in_specs=[
        pl.BlockSpec((2, _BV), lambda k: (0, k)),
        pl.BlockSpec((_EMB, _BV), lambda k: (0, k)),
    ],
    out_specs=pl.BlockSpec((1, _EMB), lambda k: (0, 0)),
    out_shape=jax.ShapeDtypeStruct((1, _EMB), jnp.float32),
)


def kernel(theta, bow, word_inputs, W1, b1, gamma, beta, W2, b2, embedding):
    score = _mlp_call(theta, bow, W1.T, gamma, beta, b1, W2.reshape(_HID), b2)

    s2 = _build_scatter()(word_inputs, bow)
    word_embedding = _bag_call(s2, embedding.T).reshape(_EMB)
    return score.reshape(1), word_embedding


# speedup vs baseline: 1.0807x; 1.0176x over previous
import functools
import math

import jax
import jax.numpy as jnp
from jax import lax
from jax.experimental import pallas as pl
from jax.experimental.pallas import tpu as pltpu
from jax.experimental.pallas import tpu_sc as plsc

_N_TOPIC = 256
_V_DIM = 10000
_HID = 2048
_EMB = 300
_VOCAB = 100000
_K = _N_TOPIC + _V_DIM

_BKK = 1024
_KGRID = 10
_BN_INV = 1.0 / math.sqrt(1.0 + 1e-5)


def _mlp_body(theta_ref, bow_ref, w1t_ref, w1tail_ref, gamma_ref, beta_ref,
              b1_ref, w2_ref, b2_ref, out_ref, p_scr, h_scr):
    k = pl.program_id(0)

    @pl.when(k == 0)
    def _concat():
        p_scr[0:1, pl.ds(0, _N_TOPIC)] = theta_ref[...][None, :]
        p_scr[0:1, pl.ds(_N_TOPIC, _V_DIM)] = bow_ref[...][None, :]

    part = lax.dot_general(
        p_scr[0:1, pl.ds(k * _BKK, _BKK)], w1t_ref[...],
        (((1,), (0,)), ((), ())), preferred_element_type=jnp.float32,
    )

    @pl.when(k == 0)
    def _init():
        h_scr[...] = part

    @pl.when(k > 0)
    def _acc():
        h_scr[...] += part

    @pl.when(k == _KGRID - 1)
    def _epilogue():
        tail = lax.dot_general(
            p_scr[0:1, pl.ds(_KGRID * _BKK, _K - _KGRID * _BKK)],
            w1tail_ref[...], (((1,), (0,)), ((), ())),
            preferred_element_type=jnp.float32,
        )
        h = h_scr[...] + tail
        scale = gamma_ref[...][None, :] * _BN_INV
        h = (h + b1_ref[...][None, :]) * scale + beta_ref[...][None, :]
        h = jnp.where(h > 0.0, h, 0.01 * h)
        out_ref[...] = (b2_ref[...][None, :]
                        + jnp.sum(h * w2_ref[...][None, :], axis=1, keepdims=True))


_mlp_call = pl.pallas_call(
    _mlp_body,
    grid=(_KGRID,),
    in_specs=[
        pl.BlockSpec((_N_TOPIC,), lambda k: (0,)),
        pl.BlockSpec((_V_DIM,), lambda k: (0,)),
        pl.BlockSpec((_BKK, _HID), lambda k: (k, 0)),
        pl.BlockSpec((_K - _KGRID * _BKK, _HID),
                     lambda k: (_KGRID * _BKK // (_K - _KGRID * _BKK), 0)),
        pl.BlockSpec((_HID,), lambda k: (0,)),
        pl.BlockSpec((_HID,), lambda k: (0,)),
        pl.BlockSpec((_HID,), lambda k: (0,)),
        pl.BlockSpec((_HID,), lambda k: (0,)),
        pl.BlockSpec((1,), lambda k: (0,)),
    ],
    out_specs=pl.BlockSpec((1, 1), lambda k: (0, 0)),
    out_shape=jax.ShapeDtypeStruct((1, 1), jnp.float32),
    scratch_shapes=[
        pltpu.VMEM((1, _K), jnp.float32),
        pltpu.VMEM((1, _HID), jnp.float32),
    ],
)

_NW = 32
_BPW = 320
_PAD_N = _NW * _BPW
_GB = 80
_GCH = _BPW // _GB
_VPAD = 102400
_STRIPE = _VPAD // 16


def _scatter_body(idx_hbm, w_hbm, out_hbm, idx_v, w_v, zero_v, s_sh, sem):
    cid = lax.axis_index("c")
    sid = lax.axis_index("s")
    wid = sid * 2 + cid
    base = wid * _BPW

    def stage(g):
        pltpu.sync_copy(idx_hbm.at[pl.ds(base + g * _GB, _GB)], idx_v.at[g])
        pltpu.sync_copy(w_hbm.at[pl.ds(base + g * _GB, _GB)], w_v.at[g])

    stage(0)

    @pl.when(wid < _NW - 1)
    def _stage_rest():
        for g in range(1, _GCH):
            stage(g)

    def zloop(j, carry):
        zero_v[pl.ds(j * 16, 16)] = jnp.zeros((16,), jnp.float32)
        return carry

    lax.fori_loop(0, _STRIPE // 16, zloop, 0)
    pltpu.sync_copy(zero_v, s_sh.at[pl.ds(sid * _STRIPE, _STRIPE)])

    plsc.subcore_barrier()

    pltpu.sync_copy(w_v.at[0], s_sh.at[idx_v.at[0]], add=True)

    @pl.when(wid < _NW - 1)
    def _scatter_rest():
        for g in range(1, _GCH):
            pltpu.sync_copy(w_v.at[g], s_sh.at[idx_v.at[g]], add=True)

    plsc.subcore_barrier()

    pltpu.sync_copy(s_sh.at[pl.ds(sid * _STRIPE, _STRIPE)],
                    out_hbm.at[cid, pl.ds(sid * _STRIPE, _STRIPE)])


@functools.lru_cache(maxsize=1)
def _build_scatter():
    return functools.partial(
        pl.kernel,
        out_type=jax.ShapeDtypeStruct((2, _VPAD), jnp.float32),
        mesh=plsc.VectorSubcoreMesh(core_axis_name="c", subcore_axis_name="s"),
        scratch_types=[
            pltpu.VMEM((_GCH, _GB), jnp.int32),
            pltpu.VMEM((_GCH, _GB), jnp.float32),
            pltpu.VMEM((_STRIPE,), jnp.float32),
            pltpu.VMEM_SHARED((_VPAD,), jnp.float32),
            pltpu.SemaphoreType.DMA,
        ],
    )(_scatter_body)


_BV = 12800
_VGRID = _VPAD // _BV


def _bag_body(s_ref, et_ref, out_ref):
    k = pl.program_id(0)
    sv = s_ref[0:1, :] + s_ref[1:2, :]
    last_valid = _VOCAB - (_VGRID - 1) * _BV

    @pl.when(k == _VGRID - 1)
    def _mask():
        col = lax.broadcasted_iota(jnp.int32, (_EMB, _BV), 1)
        et_ref[...] = jnp.where(col < last_valid, et_ref[...], 0.0)

    et = et_ref[...]
    part = lax.dot_general(
        sv, et, (((1,), (1,)), ((), ())),
        preferred_element_type=jnp.float32,
    )

    @pl.when(k == 0)
    def _init():
        out_ref[...] = part

    @pl.when(k > 0)
    def _acc():
        out_ref[...] += part


_bag_call = pl.pallas_call(
    _bag_body,
    grid=(_VGRID,),
    in_specs=[
        pl.BlockSpec((2, _BV), lambda k: (0, k)),
        pl.BlockSpec((_EMB, _BV), lambda k: (0, k)),
    ],
    out_specs=pl.BlockSpec((1, _EMB), lambda k: (0, 0)),
    out_shape=jax.ShapeDtypeStruct((1, _EMB), jnp.float32),
)


def kernel(theta, bow, word_inputs, W1, b1, gamma, beta, W2, b2, embedding):
    w1t = W1.T
    score = _mlp_call(theta, bow, w1t, w1t, gamma, beta, b1, W2.reshape(_HID), b2)

    s2 = _build_scatter()(word_inputs, bow)
    word_embedding = _bag_call(s2, embedding.T).reshape(_EMB)
    return score.reshape(1), word_embedding


# speedup vs baseline: 1.0809x; 1.0001x over previous
import functools
import math

import jax
import jax.numpy as jnp
from jax import lax
from jax.experimental import pallas as pl
from jax.experimental.pallas import tpu as pltpu
from jax.experimental.pallas import tpu_sc as plsc

_N_TOPIC = 256
_V_DIM = 10000
_HID = 2048
_EMB = 300
_VOCAB = 100000
_K = _N_TOPIC + _V_DIM

_BKK = 1024
_KGRID = 10
_BN_INV = 1.0 / math.sqrt(1.0 + 1e-5)


def _mlp_body(theta_ref, bow_ref, w1t_ref, w1tail_ref, gamma_ref, beta_ref,
              b1_ref, w2_ref, b2_ref, out_ref, p_scr, h_scr):
    k = pl.program_id(0)

    @pl.when(k == 0)
    def _concat():
        p_scr[0:1, pl.ds(0, _N_TOPIC)] = theta_ref[...][None, :]
        p_scr[0:1, pl.ds(_N_TOPIC, _V_DIM)] = bow_ref[...][None, :]

    part = lax.dot_general(
        p_scr[0:1, pl.ds(k * _BKK, _BKK)], w1t_ref[...],
        (((1,), (0,)), ((), ())), preferred_element_type=jnp.float32,
    )

    @pl.when(k == 0)
    def _init():
        h_scr[...] = part

    @pl.when(k > 0)
    def _acc():
        h_scr[...] += part

    @pl.when(k == _KGRID - 1)
    def _epilogue():
        tail = lax.dot_general(
            p_scr[0:1, pl.ds(_KGRID * _BKK, _K - _KGRID * _BKK)],
            w1tail_ref[...], (((1,), (0,)), ((), ())),
            preferred_element_type=jnp.float32,
        )
        h = h_scr[...] + tail
        scale = gamma_ref[...][None, :] * _BN_INV
        h = (h + b1_ref[...][None, :]) * scale + beta_ref[...][None, :]
        h = jnp.where(h > 0.0, h, 0.01 * h)
        out_ref[...] = (b2_ref[...][None, :]
                        + jnp.sum(h * w2_ref[...][None, :], axis=1, keepdims=True))


_mlp_call = pl.pallas_call(
    _mlp_body,
    grid=(_KGRID,),
    in_specs=[
        pl.BlockSpec((_N_TOPIC,), lambda k: (0,)),
        pl.BlockSpec((_V_DIM,), lambda k: (0,)),
        pl.BlockSpec((_BKK, _HID), lambda k: (k, 0)),
        pl.BlockSpec((_K - _KGRID * _BKK, _HID),
                     lambda k: (_KGRID * _BKK // (_K - _KGRID * _BKK), 0)),
        pl.BlockSpec((_HID,), lambda k: (0,)),
        pl.BlockSpec((_HID,), lambda k: (0,)),
        pl.BlockSpec((_HID,), lambda k: (0,)),
        pl.BlockSpec((_HID,), lambda k: (0,)),
        pl.BlockSpec((1,), lambda k: (0,)),
    ],
    out_specs=pl.BlockSpec((1, 1), lambda k: (0, 0)),
    out_shape=jax.ShapeDtypeStruct((1, 1), jnp.float32),
    scratch_shapes=[
        pltpu.VMEM((1, _K), jnp.float32),
        pltpu.VMEM((1, _HID), jnp.float32),
    ],
)

_NW = 32
_BPW = 320
_PAD_N = _NW * _BPW
_GB = 80
_GCH = _BPW // _GB
_VPAD = 102400
_STRIPE = _VPAD // 16


def _scatter_body(idx_hbm, w_hbm, out_hbm, idx_v, w_v, zero_v, s_sh, sem):
    cid = lax.axis_index("c")
    sid = lax.axis_index("s")
    wid = sid * 2 + cid
    base = wid * _BPW

    def stage(g):
        pltpu.sync_copy(idx_hbm.at[pl.ds(base + g * _GB, _GB)], idx_v.at[g])
        pltpu.sync_copy(w_hbm.at[pl.ds(base + g * _GB, _GB)], w_v.at[g])

    stage(0)

    @pl.when(wid < _NW - 1)
    def _stage_rest():
        for g in range(1, _GCH):
            stage(g)

    def zloop(j, carry):
        zero_v[pl.ds(j * 16, 16)] = jnp.zeros((16,), jnp.float32)
        return carry

    lax.fori_loop(0, _STRIPE // 16, zloop, 0)
    pltpu.sync_copy(zero_v, s_sh.at[pl.ds(sid * _STRIPE, _STRIPE)])

    plsc.subcore_barrier()

    pltpu.sync_copy(w_v.at[0], s_sh.at[idx_v.at[0]], add=True)

    @pl.when(wid < _NW - 1)
    def _scatter_rest():
        for g in range(1, _GCH):
            pltpu.sync_copy(w_v.at[g], s_sh.at[idx_v.at[g]], add=True)

    plsc.subcore_barrier()

    pltpu.sync_copy(s_sh.at[pl.ds(sid * _STRIPE, _STRIPE)],
                    out_hbm.at[cid, pl.ds(sid * _STRIPE, _STRIPE)])


@functools.lru_cache(maxsize=1)
def _build_scatter():
    return functools.partial(
        pl.kernel,
        out_type=jax.ShapeDtypeStruct((2, _VPAD), jnp.float32),
        mesh=plsc.VectorSubcoreMesh(core_axis_name="c", subcore_axis_name="s"),
        scratch_types=[
            pltpu.VMEM((_GCH, _GB), jnp.int32),
            pltpu.VMEM((_GCH, _GB), jnp.float32),
            pltpu.VMEM((_STRIPE,), jnp.float32),
            pltpu.VMEM_SHARED((_VPAD,), jnp.float32),
            pltpu.SemaphoreType.DMA,
        ],
    )(_scatter_body)


_BV = 12800
_VGRID = _VPAD // _BV


def _bag_body(s_ref, et_ref, out_ref):
    k = pl.program_id(0)
    sv = s_ref[0:1, :] + s_ref[1:2, :]
    last_valid = _VOCAB - (_VGRID - 1) * _BV

    @pl.when(k == _VGRID - 1)
    def _mask():
        col = lax.broadcasted_iota(jnp.int32, (_EMB, _BV), 1)
        et_ref[...] = jnp.where(col < last_valid, et_ref[...], 0.0)

    et = et_ref[...]
    part = lax.dot_general(
        sv, et, (((1,), (1,)), ((), ())),
        preferred_element_type=jnp.float32,
    )

    @pl.when(k == 0)
    def _init():
        out_ref[...] = part

    @pl.when(k > 0)
    def _acc():
        out_ref[...] += part


_bag_call = pl.pallas_call(
    _bag_body,
    grid=(_VGRID,),
    in_specs=[
        pl.BlockSpec((2, _BV), lambda k: (0, k)),
        pl.BlockSpec((_EMB, _BV), lambda k: (0, k)),
    ],
    out_specs=pl.BlockSpec((1, _EMB), lambda k: (0, 0)),
    out_shape=jax.ShapeDtypeStruct((1, _EMB), jnp.float32),
    compiler_params=pltpu.CompilerParams(dimension_semantics=("arbitrary",)),
)


def kernel(theta, bow, word_inputs, W1, b1, gamma, beta, W2, b2, embedding):
    w1t = W1.T
    score = _mlp_call(theta, bow, w1t, w1t, gamma, beta, b1, W2.reshape(_HID), b2)

    s2 = _build_scatter()(word_inputs, bow)
    word_embedding = _bag_call(s2, embedding.T).reshape(_EMB)
    return score.reshape(1), word_embedding
